# baseline (device time: 12813 ns/iter reference)
import jax
import jax.numpy as jnp
from jax import lax
from jax.experimental import pallas as pl
from jax.experimental.pallas import tpu as pltpu

NCHUNK = 4


def kernel(x):
    m, n = x.shape[2], x.shape[3]
    q = m // NCHUNK
    xb = x[0, 0].astype(jnp.bfloat16)

    def body(x_ref, out_ref, r1, s2, r2, sems):
        my_x = lax.axis_index("x")
        my_y = lax.axis_index("y")
        x_peer = (1 - my_x, my_y)
        y_peer = (my_x, 1 - my_y)
        hc = NCHUNK // 2
        peers_r1 = [x_peer] * hc + [y_peer] * hc
        peers_r2 = [y_peer] * hc + [x_peer] * hc
        order = [j for pair in zip(range(hc), range(hc, NCHUNK)) for j in pair]

        barrier_sem = pltpu.get_barrier_semaphore()
        for peer in (x_peer, y_peer):
            pl.semaphore_signal(
                barrier_sem, inc=1,
                device_id=peer, device_id_type=pl.DeviceIdType.MESH,
            )
        pl.semaphore_wait(barrier_sem, 2)

        rdma1 = [None] * NCHUNK
        for i in order:
            d = pltpu.make_async_remote_copy(
                src_ref=x_ref.at[pl.ds(i * q, q), :], dst_ref=r1.at[i],
                send_sem=sems.at[2 * i], recv_sem=sems.at[2 * i + 1],
                device_id=peers_r1[i], device_id_type=pl.DeviceIdType.MESH,
            )
            d.start()
            rdma1[i] = d

        rdma2 = [None] * NCHUNK
        for i in order:
            rdma1[i].wait_recv()
            s2[i] = (
                x_ref[pl.ds(i * q, q), :].astype(jnp.float32)
                + r1[i].astype(jnp.float32)
            ).astype(jnp.bfloat16)
            d = pltpu.make_async_remote_copy(
                src_ref=s2.at[i], dst_ref=r2.at[i],
                send_sem=sems.at[2 * NCHUNK + 2 * i],
                recv_sem=sems.at[2 * NCHUNK + 2 * i + 1],
                device_id=peers_r2[i], device_id_type=pl.DeviceIdType.MESH,
            )
            d.start()
            rdma2[i] = d

        for i in order:
            rdma2[i].wait_recv()
            out_ref[pl.ds(i * q, q), :] = (
                s2[i].astype(jnp.float32) + r2[i].astype(jnp.float32)
            ).astype(jnp.bfloat16)

        for i in range(NCHUNK):
            rdma1[i].wait_send()
            rdma2[i].wait_send()

    buf = lambda: pltpu.VMEM((NCHUNK, q, n), jnp.bfloat16)
    return pl.pallas_call(
        body,
        out_shape=jax.ShapeDtypeStruct((m, n), jnp.bfloat16),
        in_specs=[pl.BlockSpec(memory_space=pltpu.VMEM)],
        out_specs=pl.BlockSpec(memory_space=pltpu.VMEM),
        scratch_shapes=[
            buf(),
            buf(), buf(),
            pltpu.SemaphoreType.DMA((4 * NCHUNK,)),
        ],
        compiler_params=pltpu.CompilerParams(collective_id=0),
    )(xb)


# device time: 12323 ns/iter; 1.0398x vs baseline; 1.0398x over previous
import jax
import jax.numpy as jnp
from jax import lax
from jax.experimental import pallas as pl
from jax.experimental.pallas import tpu as pltpu

NCHUNK = 4


def kernel(x):
    m, n = x.shape[2], x.shape[3]
    q = m // NCHUNK

    def body(x_ref, out_ref, s1, r1, s2, r2, sems):
        my_x = lax.axis_index("x")
        my_y = lax.axis_index("y")
        x_peer = (1 - my_x, my_y)
        y_peer = (my_x, 1 - my_y)
        hc = NCHUNK // 2
        peers_r1 = [x_peer] * hc + [y_peer] * hc
        peers_r2 = [y_peer] * hc + [x_peer] * hc
        order = [j for pair in zip(range(hc), range(hc, NCHUNK)) for j in pair]

        barrier_sem = pltpu.get_barrier_semaphore()
        for peer in (x_peer, y_peer):
            pl.semaphore_signal(
                barrier_sem, inc=1,
                device_id=peer, device_id_type=pl.DeviceIdType.MESH,
            )
        s1[order[0]] = (
            x_ref[0, 0, order[0] * q:(order[0] + 1) * q, :].astype(jnp.bfloat16)
        )
        pl.semaphore_wait(barrier_sem, 2)

        rdma1 = [None] * NCHUNK
        for k, i in enumerate(order):
            if k > 0:
                s1[i] = x_ref[0, 0, i * q:(i + 1) * q, :].astype(jnp.bfloat16)
            d = pltpu.make_async_remote_copy(
                src_ref=s1.at[i], dst_ref=r1.at[i],
                send_sem=sems.at[2 * i], recv_sem=sems.at[2 * i + 1],
                device_id=peers_r1[i], device_id_type=pl.DeviceIdType.MESH,
            )
            d.start()
            rdma1[i] = d

        rdma2 = [None] * NCHUNK
        for i in order:
            rdma1[i].wait_recv()
            s2[i] = (
                x_ref[0, 0, i * q:(i + 1) * q, :]
                + r1[i].astype(jnp.float32)
            ).astype(jnp.bfloat16)
            d = pltpu.make_async_remote_copy(
                src_ref=s2.at[i], dst_ref=r2.at[i],
                send_sem=sems.at[2 * NCHUNK + 2 * i],
                recv_sem=sems.at[2 * NCHUNK + 2 * i + 1],
                device_id=peers_r2[i], device_id_type=pl.DeviceIdType.MESH,
            )
            d.start()
            rdma2[i] = d

        for i in order:
            rdma2[i].wait_recv()
            out_ref[i * q:(i + 1) * q, :] = (
                s2[i].astype(jnp.float32) + r2[i].astype(jnp.float32)
            ).astype(jnp.bfloat16)

        for i in range(NCHUNK):
            rdma1[i].wait_send()
            rdma2[i].wait_send()

    buf = lambda: pltpu.VMEM((NCHUNK, q, n), jnp.bfloat16)
    return pl.pallas_call(
        body,
        out_shape=jax.ShapeDtypeStruct((m, n), jnp.bfloat16),
        in_specs=[pl.BlockSpec(memory_space=pltpu.VMEM)],
        out_specs=pl.BlockSpec(memory_space=pltpu.VMEM),
        scratch_shapes=[
            buf(), buf(),
            buf(), buf(),
            pltpu.SemaphoreType.DMA((4 * NCHUNK,)),
        ],
        compiler_params=pltpu.CompilerParams(collective_id=0),
    )(x)
